# baseline (speedup 1.0000x reference)
import functools

import jax
import jax.numpy as jnp
from jax import lax
from jax.experimental import pallas as pl
from jax.experimental.pallas import tpu as pltpu
from jax.experimental.pallas import tpu_sc as plsc

N = 100000
E = 1600000
M = 128
NHID = 64
MY = 40
K = 10
ALPHA = 0.1

NHALF = N // 2
QUARTER = N // 4
NTILE = 1568
NTILE_LAST = QUARTER - 15 * NTILE
DEGPAD = 50176

CHUNK = 1024
NCHUNK = 98
EP = 16 * NCHUNK * CHUNK
EROWS = EP // 128
TROWS = EROWS // 16
TRASH = 2048
SROWS = QUARTER + TRASH

TROWS2 = EROWS // 32
PADQ = 51200
RROWS = PADQ // 128
TOTAL = 128 * PADQ

_mesh = plsc.VectorSubcoreMesh(core_axis_name="c", subcore_axis_name="s")


def _adjust_dst(dstbuf, base, co):
    iota = lax.iota(jnp.int32, 16)

    def body(v, _):
        b = v // 8
        j = v % 8
        d = dstbuf[b, pl.ds(j * 16, 16)]
        rel = d - base
        valid = (rel >= 0) & (rel < QUARTER)
        trash = QUARTER + (((co * 64 + v) & 127) * 16) + iota
        dstbuf[b, pl.ds(j * 16, 16)] = jnp.where(valid, rel, trash)
        return _

    lax.fori_loop(0, 64, body, 0, unroll=4)


def _deg_body(dst_hbm, deg_hbm, table, dbuf, sem):
    c = lax.axis_index("c")
    s = lax.axis_index("s")
    base = c * NHALF

    def zbody(i, _):
        table[pl.ds(i * 16, 16)] = jnp.zeros((16,), jnp.float32)
        return _
    lax.fori_loop(0, DEGPAD // 16, zbody, 0, unroll=4)

    ones = jnp.ones((16,), jnp.float32)

    def chunk(co, _):
        row0 = pl.multiple_of(s * TROWS + co * 8, 8)
        pltpu.async_copy(dst_hbm.at[pl.ds(row0, 8)], dbuf, sem).wait()

        def body(v, _):
            b = v // 8
            j = v % 8
            d = dbuf[b, pl.ds(j * 16, 16)]
            rel = d - base
            valid = (rel >= 0) & (rel < NHALF)
            idx = jnp.where(valid, rel, NHALF)
            plsc.addupdate_scatter(table, [idx], ones, mask=valid)
            return _

        lax.fori_loop(0, 64, body, 0, unroll=4)
        return _

    lax.fori_loop(0, NCHUNK, chunk, 0)

    wid = c * 16 + s
    off = pl.multiple_of(wid * DEGPAD, 128)
    pltpu.sync_copy(table, deg_hbm.at[pl.ds(off, DEGPAD)])


_deg_kernel = functools.partial(
    pl.kernel,
    out_type=jax.ShapeDtypeStruct((32 * DEGPAD,), jnp.float32),
    mesh=_mesh,
    scratch_types=[
        pltpu.VMEM((DEGPAD,), jnp.float32),
        pltpu.VMEM((8, 128), jnp.int32),
        pltpu.SemaphoreType.DMA,
    ],
    compiler_params=pltpu.CompilerParams(needs_layout_passes=False),
)(_deg_body)


def _part_body(src_hbm, dst_hbm, psrc_hbm, pdst_hbm, cnt_hbm,
               sbuf, dbuf, stg_s, stg_d, cbuf, sem):
    c = lax.axis_index("c")
    s = lax.axis_index("s")
    wid = c * 16 + s
    iota = lax.iota(jnp.int32, 16)
    zero16 = jnp.zeros((16,), jnp.int32)

    def flush(qt, goff):
        roff = pl.multiple_of((wid * 4 + qt) * PADQ + goff * 1024, 128)
        pltpu.sync_copy(stg_s.at[qt, pl.ds(0, 1024)],
                        psrc_hbm.at[pl.ds(roff, 1024)])
        pltpu.sync_copy(stg_d.at[qt, pl.ds(0, 1024)],
                        pdst_hbm.at[pl.ds(roff, 1024)])

    def vec(v, carry, b):
        ptrs, goffs = carry
        sv = sbuf[b, pl.ds(v * 16, 16)]
        dv = dbuf[b, pl.ds(v * 16, 16)]
        q = dv // QUARTER
        dr = dv - q * QUARTER
        nptrs = []
        ngoffs = []
        for qt in range(4):
            m = q == qt
            ptr = ptrs[qt]
            plsc.store_compressed(stg_s.at[qt, pl.ds(ptr, 16)], sv, mask=m)
            plsc.store_compressed(stg_d.at[qt, pl.ds(ptr, 16)], dr, mask=m)
            nptr = ptr + jnp.sum(m.astype(jnp.int32))
            full = nptr >= 1024

            @pl.when(full)
            def _():
                flush(qt, goffs[qt])
                ts = stg_s[qt, pl.ds(1024, 16)]
                td = stg_d[qt, pl.ds(1024, 16)]
                stg_s[qt, pl.ds(0, 16)] = ts
                stg_d[qt, pl.ds(0, 16)] = td

            nptrs.append(jnp.where(full, nptr - 1024, nptr))
            ngoffs.append(goffs[qt] + full.astype(jnp.int32))
        return tuple(nptrs), tuple(ngoffs)

    def chunk(co, carry):
        row0 = pl.multiple_of(wid * TROWS2 + co * 8, 8)
        g0 = pltpu.async_copy(src_hbm.at[pl.ds(row0, 8)], sbuf, sem)
        g1 = pltpu.async_copy(dst_hbm.at[pl.ds(row0, 8)], dbuf, sem)
        g0.wait()
        g1.wait()

        def body(v, cr):
            return vec(v % 8, cr, v // 8)

        return lax.fori_loop(0, 64, body, carry)

    zeros4 = (jnp.int32(0),) * 4
    ptrs, goffs = lax.fori_loop(0, TROWS2 // 8, chunk, (zeros4, zeros4))

    gfin = []
    for qt in range(4):
        ptr = ptrs[qt]
        sent_d = QUARTER + iota
        stg_s[qt, pl.ds(ptr, 16)] = zero16
        stg_d[qt, pl.ds(ptr, 16)] = sent_d
        for j in range(64):
            @pl.when(j * 16 >= ptr)
            def _():
                stg_s[qt, pl.ds(j * 16, 16)] = zero16
                stg_d[qt, pl.ds(j * 16, 16)] = QUARTER + ((j % 8) * 16) + iota

        @pl.when(ptr > 0)
        def _():
            flush(qt, goffs[qt])

        gfin.append(goffs[qt] + (ptrs[qt] > 0).astype(jnp.int32))

    gv = zero16
    for qt in range(4):
        gv = jnp.where(iota == qt, gfin[qt], gv)
    cbuf[pl.ds(0, 16)] = gv
    for j in range(1, 8):
        cbuf[pl.ds(j * 16, 16)] = zero16
    pltpu.sync_copy(cbuf, cnt_hbm.at[pl.ds(pl.multiple_of(wid * 128, 128), 128)])


_part_kernel = functools.partial(
    pl.kernel,
    out_type=[
        jax.ShapeDtypeStruct((TOTAL,), jnp.int32),
        jax.ShapeDtypeStruct((TOTAL,), jnp.int32),
        jax.ShapeDtypeStruct((4096,), jnp.int32),
    ],
    mesh=_mesh,
    scratch_types=[
        pltpu.VMEM((8, 128), jnp.int32),
        pltpu.VMEM((8, 128), jnp.int32),
        pltpu.VMEM((4, 1040), jnp.int32),
        pltpu.VMEM((4, 1040), jnp.int32),
        pltpu.VMEM((128,), jnp.int32),
        pltpu.SemaphoreType.DMA,
    ],
    compiler_params=pltpu.CompilerParams(needs_layout_passes=False,
                                         use_tc_tiling_on_sc=False),
)(_part_body)


def _node_rows_copy(s, base, copy_one):
    off = pl.multiple_of(base + s * NTILE, 8)
    loc = pl.multiple_of(s * NTILE, 8)
    copy_one(off, loc, NTILE_LAST)

    @pl.when(s < 15)
    def _():
        off2 = pl.multiple_of(base + s * NTILE + NTILE_LAST, 8)
        loc2 = pl.multiple_of(s * NTILE + NTILE_LAST, 8)
        copy_one(off2, loc2, NTILE - NTILE_LAST)


def _prop_body(zp_hbm, psrc_hbm, pdst_hbm, cnt_hbm, out_hbm,
               sidx, didx, rows, cntv, acc, gsem, ssem):
    c = lax.axis_index("c")
    s = lax.axis_index("s")

    iota = lax.iota(jnp.int32, 16)
    for r in range(2):
        w = 2 * s + r
        off = pl.multiple_of(w * 128, 128)
        pltpu.sync_copy(cnt_hbm.at[pl.ds(off, 128)], cntv.at[r])

    for p in range(2):
        qx = 2 * c + p
        base = c * NHALF + p * QUARTER

        def init_one(off, loc, n):
            pltpu.sync_copy(zp_hbm.at[pl.ds(off, n)], acc.at[pl.ds(loc, n)])

        _node_rows_copy(s, base, init_one)
        plsc.subcore_barrier()

        for r in range(2):
            w = 2 * s + r
            nb = jnp.sum(jnp.where(iota == qx, cntv[r, pl.ds(0, 16)], 0))
            roffr = (w * 4 + qx) * RROWS

            def group(g, _):
                row0 = pl.multiple_of(roffr + g * 8, 8)
                g0 = pltpu.async_copy(psrc_hbm.at[pl.ds(row0, 8)], sidx, gsem)
                g1 = pltpu.async_copy(pdst_hbm.at[pl.ds(row0, 8)], didx, gsem)
                g0.wait()
                g1.wait()
                descs = []
                for b in range(8):
                    descs.append(pltpu.async_copy(
                        zp_hbm.at[sidx.at[b]], rows.at[pl.ds(b * 128, 128)],
                        gsem))
                for d in descs:
                    d.wait()
                descs = []
                for b in range(8):
                    descs.append(pltpu.async_copy(
                        rows.at[pl.ds(b * 128, 128)], acc.at[didx.at[b]],
                        ssem, add=True))
                for d in descs:
                    d.wait()
                return _

            lax.fori_loop(0, nb, group, 0)

        plsc.subcore_barrier()

        def flush_one(off, loc, n):
            pltpu.sync_copy(acc.at[pl.ds(loc, n)], out_hbm.at[pl.ds(off, n)])

        _node_rows_copy(s, base, flush_one)
        plsc.subcore_barrier()


_prop_kernel = functools.partial(
    pl.kernel,
    out_type=jax.ShapeDtypeStruct((N, MY), jnp.float32),
    mesh=_mesh,
    scratch_types=[
        pltpu.VMEM((8, 128), jnp.int32),
        pltpu.VMEM((8, 128), jnp.int32),
        pltpu.VMEM((CHUNK, MY), jnp.float32),
        pltpu.VMEM((2, 128), jnp.int32),
        pltpu.VMEM_SHARED((SROWS, MY), jnp.float32),
        pltpu.SemaphoreType.DMA,
        pltpu.SemaphoreType.DMA,
    ],
    compiler_params=pltpu.CompilerParams(needs_layout_passes=False,
                                         use_tc_tiling_on_sc=False),
)(_prop_body)


ROW_BLK = 2000


def _consts_body(x_ref, w1t_ref, b1_ref, w2t_ref, b2_ref, deg_ref,
                 h_ref, z0p_ref, c1_ref, bf_ref):
    d = jnp.sum(deg_ref[...], axis=1, keepdims=True) + 1.0
    dinv = lax.rsqrt(d)
    h = jnp.maximum(x_ref[...] @ w1t_ref[...] + b1_ref[...], 0.0)
    h = h @ w2t_ref[...] + b2_ref[...]
    h_ref[...] = h
    z0p_ref[...] = dinv * h
    c1_ref[...] = (1.0 - ALPHA) / d
    bf_ref[...] = (1.0 - ALPHA) * dinv


def _consts(x, W1, b1, W2, b2, deg_raw):
    return pl.pallas_call(
        _consts_body,
        grid=(N // ROW_BLK,),
        in_specs=[
            pl.BlockSpec((ROW_BLK, M), lambda i: (i, 0)),
            pl.BlockSpec((M, NHID), lambda i: (0, 0)),
            pl.BlockSpec((1, NHID), lambda i: (0, 0)),
            pl.BlockSpec((NHID, MY), lambda i: (0, 0)),
            pl.BlockSpec((1, MY), lambda i: (0, 0)),
            pl.BlockSpec((ROW_BLK, 16), lambda i: (i, 0)),
        ],
        out_specs=[
            pl.BlockSpec((ROW_BLK, MY), lambda i: (i, 0)),
            pl.BlockSpec((ROW_BLK, MY), lambda i: (i, 0)),
            pl.BlockSpec((ROW_BLK, 1), lambda i: (i, 0)),
            pl.BlockSpec((ROW_BLK, 1), lambda i: (i, 0)),
        ],
        out_shape=[
            jax.ShapeDtypeStruct((N, MY), jnp.float32),
            jax.ShapeDtypeStruct((N, MY), jnp.float32),
            jax.ShapeDtypeStruct((N, 1), jnp.float32),
            jax.ShapeDtypeStruct((N, 1), jnp.float32),
        ],
    )(x, W1.T, b1[None, :], W2.T, b2[None, :], deg_raw)


def _axpb_body(s_ref, a_ref, b_ref, o_ref):
    o_ref[...] = a_ref[...] * s_ref[...] + ALPHA * b_ref[...]


def _axpb(S, a_col, B):
    return pl.pallas_call(
        _axpb_body,
        grid=(N // ROW_BLK,),
        in_specs=[
            pl.BlockSpec((ROW_BLK, MY), lambda i: (i, 0)),
            pl.BlockSpec((ROW_BLK, 1), lambda i: (i, 0)),
            pl.BlockSpec((ROW_BLK, MY), lambda i: (i, 0)),
        ],
        out_specs=pl.BlockSpec((ROW_BLK, MY), lambda i: (i, 0)),
        out_shape=jax.ShapeDtypeStruct((N, MY), jnp.float32),
    )(S, a_col, B)


def kernel(x, edge_index, W1, b1, W2, b2):
    src = edge_index[0]
    dst = edge_index[1]
    src2d = jnp.pad(src, (0, EP - E)).reshape(EROWS, 128)
    dst2d = jnp.pad(dst, (0, EP - E), constant_values=N).reshape(EROWS, 128)

    psrc_f, pdst_f, cnts = _part_kernel(src2d, dst2d)
    psrc = psrc_f.reshape(TOTAL // 128, 128)
    pdst = pdst_f.reshape(TOTAL // 128, 128)

    deg_flat = _deg_kernel(dst2d)
    deg_t = (deg_flat.reshape(2, 16, DEGPAD)[:, :, :NHALF]
             .transpose(0, 2, 1).reshape(N, 16))
    h, z0p, c1, bf = _consts(x, W1, b1, W2, b2, deg_t)

    zp = z0p
    for k in range(K):
        S = _prop_kernel(zp, psrc, pdst, cnts)
        if k < K - 1:
            zp = _axpb(S, c1, z0p)
        else:
            zp = _axpb(S, bf, h)
    return zp
